# baseline (speedup 1.0000x reference)
import functools

import jax
import jax.numpy as jnp
from jax import lax
from jax.experimental import pallas as pl
from jax.experimental.pallas import tpu as pltpu
from jax.experimental.pallas import tpu_sc as plsc

BATCH = 16384
EMBED_DIM = 64
HIDDEN_DIM = 128
OUTPUT_DIM = 64

_NC = 2
_NS = 16
_NW = _NC * _NS
_B_PER_W = BATCH // _NW


def _make_sc_gather():
    mesh = plsc.VectorSubcoreMesh(core_axis_name="c", subcore_axis_name="s")

    @functools.partial(
        pl.kernel,
        mesh=mesh,
        out_type=jax.ShapeDtypeStruct((BATCH, EMBED_DIM), jnp.float32),
        scratch_types=[
            pltpu.VMEM((_B_PER_W,), jnp.int32),
            pltpu.VMEM((_B_PER_W, EMBED_DIM), jnp.float32),
            pltpu.SemaphoreType.DMA,
        ],
        compiler_params=pltpu.CompilerParams(use_tc_tiling_on_sc=False),
    )
    def gather_k(table_hbm, idx_hbm, out_hbm, idx_v, rows_v, sem):
        wid = lax.axis_index("s") * _NC + lax.axis_index("c")
        base = wid * _B_PER_W
        pltpu.sync_copy(idx_hbm.at[pl.ds(base, _B_PER_W)], idx_v)
        pltpu.async_copy(table_hbm.at[idx_v], rows_v, sem).wait()
        pltpu.sync_copy(rows_v, out_hbm.at[pl.ds(base, _B_PER_W)])

    return gather_k


_sc_gather = _make_sc_gather()

_MLP_BLK = 2048


def _mlp_body(x_ref, w1t_ref, b1_ref, w2t_ref, b2_ref, o_ref):
    x = x_ref[...]
    h = jnp.dot(x, w1t_ref[...], preferred_element_type=jnp.float32)
    h = jnp.maximum(h + b1_ref[...], 0.0)
    o = jnp.dot(h, w2t_ref[...], preferred_element_type=jnp.float32)
    o_ref[...] = o + b2_ref[...]


def _mlp(x, w1t, b1, w2t, b2):
    n = x.shape[0]
    grid = (n // _MLP_BLK,)
    return pl.pallas_call(
        _mlp_body,
        grid=grid,
        in_specs=[
            pl.BlockSpec((_MLP_BLK, EMBED_DIM), lambda i: (i, 0)),
            pl.BlockSpec((EMBED_DIM, HIDDEN_DIM), lambda i: (0, 0)),
            pl.BlockSpec((1, HIDDEN_DIM), lambda i: (0, 0)),
            pl.BlockSpec((HIDDEN_DIM, OUTPUT_DIM), lambda i: (0, 0)),
            pl.BlockSpec((1, OUTPUT_DIM), lambda i: (0, 0)),
        ],
        out_specs=pl.BlockSpec((_MLP_BLK, OUTPUT_DIM), lambda i: (i, 0)),
        out_shape=jax.ShapeDtypeStruct((n, OUTPUT_DIM), jnp.float32),
    )(x, w1t, b1, w2t, b2)


def kernel(condition, table, W1, b1, W2, b2):
    idx = condition.astype(jnp.int32)
    rows = _sc_gather(table, idx)
    return _mlp(rows, W1.T, b1.reshape(1, -1), W2.T, b2.reshape(1, -1))


# speedup vs baseline: 1.0011x; 1.0011x over previous
import functools

import jax
import jax.numpy as jnp
from jax import lax
from jax.experimental import pallas as pl
from jax.experimental.pallas import tpu as pltpu
from jax.experimental.pallas import tpu_sc as plsc

NUM_CLASSES = 1000000
BATCH = 16384
EMBED_DIM = 64
HIDDEN_DIM = 128
OUTPUT_DIM = 64

_GATHER_W = 2 * EMBED_DIM

_NC = 2
_NS = 16
_NW = _NC * _NS
_B_PER_W = BATCH // _NW


def _make_sc_gather():
    mesh = plsc.VectorSubcoreMesh(core_axis_name="c", subcore_axis_name="s")

    @functools.partial(
        pl.kernel,
        mesh=mesh,
        out_type=jax.ShapeDtypeStruct((BATCH, _GATHER_W), jnp.float32),
        scratch_types=[
            pltpu.VMEM((_B_PER_W,), jnp.int32),
            pltpu.VMEM((_B_PER_W, _GATHER_W), jnp.float32),
            pltpu.SemaphoreType.DMA,
        ],
    )
    def gather_k(table_hbm, idx_hbm, out_hbm, idx_v, rows_v, sem):
        wid = lax.axis_index("s") * _NC + lax.axis_index("c")
        base = wid * _B_PER_W
        pltpu.sync_copy(idx_hbm.at[pl.ds(base, _B_PER_W)], idx_v)
        pltpu.async_copy(table_hbm.at[idx_v], rows_v, sem).wait()
        pltpu.sync_copy(rows_v, out_hbm.at[pl.ds(base, _B_PER_W)])

    return gather_k


_sc_gather = _make_sc_gather()

_MLP_BLK = 2048


def _mlp_body(x2_ref, p_ref, w1t_ref, b1_ref, w2t_ref, b2_ref, o_ref):
    x2 = x2_ref[...]
    p = p_ref[...]
    x = x2[:, :EMBED_DIM] * (1.0 - p) + x2[:, EMBED_DIM:] * p
    h = jnp.dot(x, w1t_ref[...], preferred_element_type=jnp.float32)
    h = jnp.maximum(h + b1_ref[...], 0.0)
    o = jnp.dot(h, w2t_ref[...], preferred_element_type=jnp.float32)
    o_ref[...] = o + b2_ref[...]


def _mlp(x2, parf, w1t, b1, w2t, b2):
    n = x2.shape[0]
    grid = (n // _MLP_BLK,)
    return pl.pallas_call(
        _mlp_body,
        grid=grid,
        in_specs=[
            pl.BlockSpec((_MLP_BLK, _GATHER_W), lambda i: (i, 0)),
            pl.BlockSpec((_MLP_BLK, 1), lambda i: (i, 0)),
            pl.BlockSpec((EMBED_DIM, HIDDEN_DIM), lambda i: (0, 0)),
            pl.BlockSpec((1, HIDDEN_DIM), lambda i: (0, 0)),
            pl.BlockSpec((HIDDEN_DIM, OUTPUT_DIM), lambda i: (0, 0)),
            pl.BlockSpec((1, OUTPUT_DIM), lambda i: (0, 0)),
        ],
        out_specs=pl.BlockSpec((_MLP_BLK, OUTPUT_DIM), lambda i: (i, 0)),
        out_shape=jax.ShapeDtypeStruct((n, OUTPUT_DIM), jnp.float32),
    )(x2, parf, w1t, b1, w2t, b2)


def kernel(condition, table, W1, b1, W2, b2):
    idx = condition.astype(jnp.int32)
    idx2 = idx >> 1
    parf = (idx & 1).astype(jnp.float32).reshape(-1, 1)
    t2 = table.reshape(NUM_CLASSES // 2, _GATHER_W)
    x2 = _sc_gather(t2, idx2)
    return _mlp(x2, parf, W1.T, b1.reshape(1, -1), W2.T, b2.reshape(1, -1))


# speedup vs baseline: 1.7124x; 1.7106x over previous
import functools

import jax
import jax.numpy as jnp
from jax import lax
from jax.experimental import pallas as pl
from jax.experimental.pallas import tpu as pltpu
from jax.experimental.pallas import tpu_sc as plsc

NUM_CLASSES = 1000000
BATCH = 16384
EMBED_DIM = 64
HIDDEN_DIM = 128
OUTPUT_DIM = 64

_NC = 2
_NS = 16
_NW = _NC * _NS
_B_PER_W = BATCH // _NW


def _make_sc_gather():
    mesh = plsc.VectorSubcoreMesh(core_axis_name="c", subcore_axis_name="s")

    @functools.partial(
        pl.kernel,
        mesh=mesh,
        out_type=jax.ShapeDtypeStruct((BATCH, EMBED_DIM), jnp.float32),
        scratch_types=[
            pltpu.VMEM((_B_PER_W,), jnp.int32),
            pltpu.VMEM((_B_PER_W, EMBED_DIM), jnp.float32),
            pltpu.SemaphoreType.DMA,
        ],
    )
    def gather_k(table_hbm, idx_hbm, out_hbm, idx_v, rows_v, sem):
        wid = lax.axis_index("s") * _NC + lax.axis_index("c")
        base = wid * _B_PER_W
        pltpu.sync_copy(idx_hbm.at[pl.ds(base, _B_PER_W)], idx_v)

        def issue(g, carry):
            v = idx_v[pl.ds(g * 16, 16)]
            for l in range(16):
                i = v[l]
                pltpu.async_copy(table_hbm.at[i], rows_v.at[g * 16 + l], sem)
            return carry

        lax.fori_loop(0, _B_PER_W // 16, issue, 0)
        pltpu.make_async_copy(
            table_hbm.at[pl.ds(0, _B_PER_W)], rows_v, sem
        ).wait()
        pltpu.sync_copy(rows_v, out_hbm.at[pl.ds(base, _B_PER_W)])

    return gather_k


_sc_gather = _make_sc_gather()

_MLP_BLK = 2048


def _mlp_body(x_ref, w1t_ref, b1_ref, w2t_ref, b2_ref, o_ref):
    x = x_ref[...]
    h = jnp.dot(x, w1t_ref[...], preferred_element_type=jnp.float32)
    h = jnp.maximum(h + b1_ref[...], 0.0)
    o = jnp.dot(h, w2t_ref[...], preferred_element_type=jnp.float32)
    o_ref[...] = o + b2_ref[...]


def _mlp(x, w1t, b1, w2t, b2):
    n = x.shape[0]
    grid = (n // _MLP_BLK,)
    return pl.pallas_call(
        _mlp_body,
        grid=grid,
        in_specs=[
            pl.BlockSpec((_MLP_BLK, EMBED_DIM), lambda i: (i, 0)),
            pl.BlockSpec((EMBED_DIM, HIDDEN_DIM), lambda i: (0, 0)),
            pl.BlockSpec((1, HIDDEN_DIM), lambda i: (0, 0)),
            pl.BlockSpec((HIDDEN_DIM, OUTPUT_DIM), lambda i: (0, 0)),
            pl.BlockSpec((1, OUTPUT_DIM), lambda i: (0, 0)),
        ],
        out_specs=pl.BlockSpec((_MLP_BLK, OUTPUT_DIM), lambda i: (i, 0)),
        out_shape=jax.ShapeDtypeStruct((n, OUTPUT_DIM), jnp.float32),
    )(x, w1t, b1, w2t, b2)


def kernel(condition, table, W1, b1, W2, b2):
    idx = condition.astype(jnp.int32)
    rows = _sc_gather(table, idx)
    return _mlp(rows, W1.T, b1.reshape(1, -1), W2.T, b2.reshape(1, -1))


# speedup vs baseline: 1.7466x; 1.0200x over previous
import functools

import jax
import jax.numpy as jnp
from jax import lax
from jax.experimental import pallas as pl
from jax.experimental.pallas import tpu as pltpu
from jax.experimental.pallas import tpu_sc as plsc

NUM_CLASSES = 1000000
BATCH = 16384
EMBED_DIM = 64
HIDDEN_DIM = 128
OUTPUT_DIM = 64

_NC = 2
_NS = 16
_NW = _NC * _NS
_B_PER_W = BATCH // _NW


def _make_sc_gather():
    mesh = plsc.VectorSubcoreMesh(core_axis_name="c", subcore_axis_name="s")

    @functools.partial(
        pl.kernel,
        mesh=mesh,
        out_type=jax.ShapeDtypeStruct((BATCH, EMBED_DIM), jnp.float32),
        scratch_types=[
            pltpu.VMEM((_B_PER_W,), jnp.int32),
            pltpu.VMEM((_B_PER_W, EMBED_DIM), jnp.float32),
            pltpu.SemaphoreType.DMA,
        ],
    )
    def gather_k(table_hbm, idx_hbm, out_hbm, idx_v, rows_v, sem):
        wid = lax.axis_index("s") * _NC + lax.axis_index("c")
        base = wid * _B_PER_W
        pltpu.sync_copy(idx_hbm.at[pl.ds(base, _B_PER_W)], idx_v)

        def issue(g, carry):
            v = idx_v[pl.ds(g * 16, 16)]
            for l in range(16):
                pltpu.async_copy(
                    table_hbm.at[v[l]], rows_v.at[g * 16 + l], sem
                )
            return carry

        lax.fori_loop(0, _B_PER_W // 16, issue, 0)
        pltpu.make_async_copy(
            out_hbm.at[pl.ds(base, _B_PER_W)], rows_v, sem
        ).wait()
        pltpu.sync_copy(rows_v, out_hbm.at[pl.ds(base, _B_PER_W)])

    return gather_k


_sc_gather = _make_sc_gather()

_MLP_BLK = 2048


def _mlp_body(x_ref, w1t_ref, b1_ref, w2t_ref, b2_ref, ot_ref):
    x = x_ref[...]
    h = jnp.dot(x, w1t_ref[...], preferred_element_type=jnp.float32)
    h = jnp.maximum(h + b1_ref[...], 0.0)
    o = jnp.dot(h, w2t_ref[...], preferred_element_type=jnp.float32)
    ot_ref[...] = (o + b2_ref[...]).T


def _mlp(x, w1t, b1, w2t, b2):
    n = x.shape[0]
    grid = (n // _MLP_BLK,)
    return pl.pallas_call(
        _mlp_body,
        grid=grid,
        in_specs=[
            pl.BlockSpec((_MLP_BLK, EMBED_DIM), lambda i: (i, 0)),
            pl.BlockSpec((EMBED_DIM, HIDDEN_DIM), lambda i: (0, 0)),
            pl.BlockSpec((1, HIDDEN_DIM), lambda i: (0, 0)),
            pl.BlockSpec((HIDDEN_DIM, OUTPUT_DIM), lambda i: (0, 0)),
            pl.BlockSpec((1, OUTPUT_DIM), lambda i: (0, 0)),
        ],
        out_specs=pl.BlockSpec((OUTPUT_DIM, _MLP_BLK), lambda i: (0, i)),
        out_shape=jax.ShapeDtypeStruct((OUTPUT_DIM, n), jnp.float32),
    )(x, w1t, b1, w2t, b2)


def kernel(condition, table, W1, b1, W2, b2):
    idx = condition.astype(jnp.int32)
    rows = _sc_gather(table, idx)
    ot = _mlp(rows, W1.T, b1.reshape(1, -1), W2.T, b2.reshape(1, -1))
    return ot.T
